# baseline (speedup 1.0000x reference)
import functools

import numpy as np

import jax
import jax.numpy as jnp
from jax import lax
from jax.experimental import pallas as pl
from jax.experimental.pallas import tpu as pltpu
from jax.experimental.pallas import tpu_sc as plsc

N_GRAPHS_C = 16384
N_NODES_C = 10
N_EDGES_C = 50
D_IN_C = 4
D_HID_C = 16
N_CLASSES_C = 5

NC = 2
NS = 16
LANES = 16

NW = NC * NS
GPW = N_GRAPHS_C // NW
GPC = 128
NCHUNK = GPW // GPC
NGC = GPC // LANES

ROW = 128
XW = N_NODES_C * D_IN_C
EW = 2 * N_EDGES_C
X_CHUNK = GPC * XW
E_CHUNK = GPC * EW
Y_CHUNK = GPC * ROW
NODES_G = LANES * N_NODES_C

_REPACK_PATS = []
for _k in range(5):
    _w = np.arange(16) + 16 * _k
    _REPACK_PATS.append(((_w // XW) * ROW + _w % XW).astype(np.int32))


def _sc_aggregate(x_flat, e_flat, table):
    mesh = plsc.VectorSubcoreMesh(
        core_axis_name="c", subcore_axis_name="s",
        num_cores=NC, num_subcores=NS)

    @functools.partial(
        pl.kernel,
        out_type=jax.ShapeDtypeStruct((N_GRAPHS_C * ROW,), jnp.float32),
        mesh=mesh,
        scratch_types=[
            pltpu.VMEM((64,), jnp.float32),
            pltpu.VMEM((X_CHUNK,), jnp.float32),
            pltpu.VMEM((E_CHUNK,), jnp.int32),
            pltpu.VMEM((X_CHUNK,), jnp.float32),
            pltpu.VMEM((Y_CHUNK,), jnp.float32),
            pltpu.VMEM((NODES_G,), jnp.float32),
            pltpu.VMEM((NODES_G,), jnp.float32),
        ],
        compiler_params=pltpu.CompilerParams(needs_layout_passes=False),
    )
    def agg(x_hbm, e_hbm, t_hbm, y_hbm, tab, xb, eb, yb, yr, deg, dnv):
        wid = lax.axis_index("s") * NC + lax.axis_index("c")
        pltpu.sync_copy(t_hbm, tab)
        iota = lax.iota(jnp.int32, LANES)
        offs = iota * N_NODES_C
        ones = jnp.ones((LANES,), jnp.float32)
        zeros = jnp.zeros((LANES,), jnp.float32)
        pats = []
        for k in range(5):
            w = iota + 16 * k
            pats.append(w + jnp.where(w >= XW, ROW - XW, 0))

        def chunk_body(ci, _):
            g0 = wid * GPW + ci * GPC
            pltpu.sync_copy(x_hbm.at[pl.ds(g0 * XW, X_CHUNK)], xb)
            pltpu.sync_copy(e_hbm.at[pl.ds(g0 * EW, E_CHUNK)], eb)

            def group_body(gi, _):
                eb0 = gi * (LANES * EW)
                xob = gi * (LANES * XW)
                elane = iota * EW + eb0
                xlane = iota * XW + xob

                for t in range(N_NODES_C):
                    deg[pl.ds(t * 16, 16)] = zeros

                def deg_body(j):
                    dd = plsc.load_gather(eb, [elane + (N_EDGES_C + j)])
                    plsc.addupdate_scatter(deg, [dd + offs], ones)
                plsc.parallel_loop(0, N_EDGES_C, 1, unroll=10)(deg_body)

                def dinv_body(t):
                    dv = deg[pl.ds(t * 16, 16)] + 1.0
                    di = dv.astype(jnp.int32)
                    r = plsc.load_gather(tab, [di])
                    dnv[pl.ds(t * 16, 16)] = r
                    r2 = r * r
                    x4 = xob + (t * 16 + iota) * D_IN_C
                    for c in range(D_IN_C):
                        xv = plsc.load_gather(xb, [x4 + c])
                        plsc.store_scatter(yb, [x4 + c], xv * r2)
                plsc.parallel_loop(0, N_NODES_C, 1, unroll=5)(dinv_body)

                def main_body(j):
                    ss = plsc.load_gather(eb, [elane + j])
                    dd = plsc.load_gather(eb, [elane + (N_EDGES_C + j)])
                    nrm = (plsc.load_gather(dnv, [ss + offs])
                           * plsc.load_gather(dnv, [dd + offs]))
                    xs = xlane + ss * D_IN_C
                    yd = xlane + dd * D_IN_C
                    for c in range(D_IN_C):
                        xv = plsc.load_gather(xb, [xs + c])
                        plsc.addupdate_scatter(yb, [yd + c], xv * nrm)
                plsc.parallel_loop(0, N_EDGES_C, 1, unroll=5)(main_body)
                return 0
            lax.fori_loop(0, NGC, group_body, 0)

            def repack_body(m):
                src = m * (2 * XW)
                dst = m * (2 * ROW)
                for k in range(5):
                    v = yb[pl.ds(src + 16 * k, 16)]
                    plsc.store_scatter(yr, [dst + pats[k]], v)
            plsc.parallel_loop(0, GPC // 2, 1, unroll=4)(repack_body)

            pltpu.sync_copy(yr, y_hbm.at[pl.ds(g0 * ROW, Y_CHUNK)])
            return 0
        lax.fori_loop(0, NCHUNK, chunk_body, 0)

    return agg(x_flat, e_flat, table)


def _tc_body(y_ref, wc_ref, bc_ref, wl_ref, bl_ref, out_ref):
    y2 = y_ref[:, :XW]
    h = jnp.dot(y2, wc_ref[...], preferred_element_type=jnp.float32)
    h = jnp.maximum(h + bc_ref[...], 0.0)
    lg = jnp.dot(h, wl_ref[...], preferred_element_type=jnp.float32)
    lg = lg + bl_ref[...]
    m = jnp.max(lg, axis=1, keepdims=True)
    e = jnp.exp(lg - m)
    s = jnp.sum(e, axis=1, keepdims=True)
    out_ref[...] = (lg - m) - jnp.log(s)


def _tc_head(y_rows, wc_big, bc_big, wl_t, bl):
    B = 2048
    grid = (N_GRAPHS_C // B,)
    return pl.pallas_call(
        _tc_body,
        grid=grid,
        in_specs=[
            pl.BlockSpec((B, ROW), lambda i: (i, 0)),
            pl.BlockSpec((XW, N_NODES_C * D_HID_C), lambda i: (0, 0)),
            pl.BlockSpec((1, N_NODES_C * D_HID_C), lambda i: (0, 0)),
            pl.BlockSpec((N_NODES_C * D_HID_C, N_CLASSES_C), lambda i: (0, 0)),
            pl.BlockSpec((1, N_CLASSES_C), lambda i: (0, 0)),
        ],
        out_specs=pl.BlockSpec((B, N_CLASSES_C), lambda i: (i, 0)),
        out_shape=jax.ShapeDtypeStruct((N_GRAPHS_C, N_CLASSES_C), jnp.float32),
    )(y_rows, wc_big, bc_big, wl_t, bl)


@jax.jit
def kernel(x_batch, edge_index_batch, W_conv, b_conv, W_lin, b_lin):
    ar = jnp.arange(64, dtype=jnp.float32)
    table = jnp.where(ar > 0, 1.0 / jnp.sqrt(jnp.maximum(ar, 1.0)), 0.0)
    wc_big = jnp.kron(jnp.eye(N_NODES_C, dtype=jnp.float32), W_conv)
    bc_big = jnp.tile(b_conv, N_NODES_C).reshape(1, -1)

    xq = jax.lax.optimization_barrier(x_batch.reshape(N_GRAPHS_C, XW))
    eq = jax.lax.optimization_barrier(edge_index_batch.reshape(N_GRAPHS_C, EW))
    xc = jax.lax.optimization_barrier(xq.reshape(N_GRAPHS_C * XW // 128, 128))
    ec = jax.lax.optimization_barrier(eq.reshape(N_GRAPHS_C * EW // 128, 128))

    y_flat = _sc_aggregate(xc.reshape(-1), ec.reshape(-1), table)
    y_rows = y_flat.reshape(N_GRAPHS_C, ROW)
    return _tc_head(y_rows, wc_big, bc_big, W_lin.T, b_lin.reshape(1, -1))
